# baseline (speedup 1.0000x reference)
import jax
import jax.numpy as jnp
from jax import lax
from jax.experimental import pallas as pl
from jax.experimental.pallas import tpu as pltpu
from jax.experimental.pallas import tpu_sc as plsc

GRID = 256
NPTS = 2097152
CH = 4
L = 16

_info = plsc.get_sparse_core_info()
_NC, _NS = _info.num_cores, _info.num_subcores
NW = _NC * _NS
PTS_PER_TILE = NPTS // NW

CHUNK = 512
NCHUNK = PTS_PER_TILE // CHUNK
NGRP = CHUNK // L
GSLICE = 128
NGS = CHUNK // GSLICE
NTAB = 8
XY_OFF = (0, 128, 32768, 32896)


def _tile_body(pts_hbm, vg_hbm, out_hbm, pts_v, idx_v, fr_v, par_v, gat_v, out_v, sem):
    wid = lax.axis_index("s") * _NC + lax.axis_index("c")
    base = wid * PTS_PER_TILE
    lanes = lax.iota(jnp.int32, L)
    zero = jnp.zeros((L,), jnp.int32)

    def chunk_body(ci, carry):
        off = base + ci * CHUNK
        pltpu.sync_copy(pts_hbm.at[pl.ds(off * 3, CHUNK * 3)], pts_v)

        def idx_body(g, c2):
            s = g * L
            rows3 = (s + lanes) * 3
            px = plsc.load_gather(pts_v, [rows3])
            py = plsc.load_gather(pts_v, [rows3 + 1])
            pz = plsc.load_gather(pts_v, [rows3 + 2])
            gx = px * 127.5 + 127.5
            gy = py * 127.5 + 127.5
            gz = pz * 127.5 + 127.5
            x0 = gx.astype(jnp.int32)
            y0 = gy.astype(jnp.int32)
            z0 = gz.astype(jnp.int32)
            fx = gx - x0.astype(jnp.float32)
            fy = gy - y0.astype(jnp.float32)
            fz = gz - z0.astype(jnp.float32)
            x0 = jnp.minimum(jnp.maximum(x0, 0), GRID - 2)
            y0 = jnp.minimum(jnp.maximum(y0, 0), GRID - 2)
            z0 = jnp.minimum(jnp.maximum(z0, 0), GRID - 2)
            f = (x0 << 16) | (y0 << 8) | z0
            par = f & 1
            ra = f >> 1
            rb = ra + par
            sl = pl.ds(s, L)
            for t in range(4):
                idx_v[2 * t, sl] = ra + XY_OFF[t]
                idx_v[2 * t + 1, sl] = rb + XY_OFF[t]
            fr_v[0, sl] = fx
            fr_v[1, sl] = fy
            fr_v[2, sl] = fz
            par_v[sl] = par << 2
            return c2

        lax.fori_loop(0, NGRP, idx_body, 0)

        copies = []
        for t in range(NTAB):
            for j in range(NGS):
                copies.append(
                    pltpu.async_copy(
                        vg_hbm.at[idx_v.at[t, pl.ds(j * GSLICE, GSLICE)]],
                        gat_v.at[t, pl.ds(j * GSLICE, GSLICE)],
                        sem,
                    )
                )
        for cp in copies:
            cp.wait()

        def comb_body(g, c2):
            s = g * L
            rows = s + lanes
            sl = pl.ds(s, L)
            fx = fr_v[0, sl]
            fy = fr_v[1, sl]
            fz = fr_v[2, sl]
            par4 = par_v[sl]
            ex = 1.0 - fx
            ey = 1.0 - fy
            ez = 1.0 - fz
            wxy = (ex * ey, ex * fy, fx * ey, fx * fy)
            wa = tuple(w * ez for w in wxy)
            wb = tuple(w * fz for w in wxy)
            rows4 = rows << 2
            for ch in range(CH):
                sa = par4 + ch
                sb = (zero + (4 + ch)) - par4
                acc = wa[0] * plsc.load_gather(gat_v.at[0], [rows, sa])
                acc = acc + wb[0] * plsc.load_gather(gat_v.at[1], [rows, sb])
                for t in range(1, 4):
                    acc = acc + wa[t] * plsc.load_gather(gat_v.at[2 * t], [rows, sa])
                    acc = acc + wb[t] * plsc.load_gather(
                        gat_v.at[2 * t + 1], [rows, sb]
                    )
                plsc.store_scatter(out_v, [rows4 + ch], acc)
            return c2

        lax.fori_loop(0, NGRP, comb_body, 0)
        pltpu.sync_copy(out_v, out_hbm.at[pl.ds(off * 4, CHUNK * 4)])
        return carry

    lax.fori_loop(0, NCHUNK, chunk_body, 0)


@jax.jit
def kernel(points, voxel_grid):
    vg = voxel_grid.reshape(GRID * GRID * GRID // 2, 2 * CH)
    pts = points.reshape(NPTS * 3)
    run = pl.kernel(
        _tile_body,
        out_type=jax.ShapeDtypeStruct((NPTS * 4,), jnp.float32),
        mesh=plsc.VectorSubcoreMesh(core_axis_name="c", subcore_axis_name="s"),
        scratch_types=[
            pltpu.VMEM((CHUNK * 3,), jnp.float32),
            pltpu.VMEM((NTAB, CHUNK), jnp.int32),
            pltpu.VMEM((3, CHUNK), jnp.float32),
            pltpu.VMEM((CHUNK,), jnp.int32),
            pltpu.VMEM((NTAB, CHUNK, 2 * CH), jnp.float32),
            pltpu.VMEM((CHUNK * 4,), jnp.float32),
            pltpu.SemaphoreType.DMA,
        ],
        compiler_params=pltpu.CompilerParams(
            needs_layout_passes=False, use_tc_tiling_on_sc=False
        ),
    )
    return run(pts, vg).reshape(NPTS, CH)


# speedup vs baseline: 6.6051x; 6.6051x over previous
import jax
import jax.numpy as jnp
from jax import lax
from jax.experimental import pallas as pl
from jax.experimental.pallas import tpu as pltpu
from jax.experimental.pallas import tpu_sc as plsc

GRID = 256
NPTS = 2097152
CH = 4
L = 16

_info = plsc.get_sparse_core_info()
_NC, _NS = _info.num_cores, _info.num_subcores
NW = _NC * _NS
PTS_PER_TILE = NPTS // NW

_CP = pltpu.CompilerParams(needs_layout_passes=False, use_tc_tiling_on_sc=False)
_MESH = dict(core_axis_name="c", subcore_axis_name="s")

NFLOAT = GRID * GRID * GRID * CH
NBLK = NFLOAT // 512
BLK_PER_TILE = NBLK // NW
K1_CHUNK = 64
K1_NCHUNK = BLK_PER_TILE // K1_CHUNK
K1_NGRP = K1_CHUNK * 32


def _fmt_body(av_hbm, t_hbm, in_v, out_v):
    wid = lax.axis_index("s") * _NC + lax.axis_index("c")
    base = wid * BLK_PER_TILE * 512
    lanes = lax.iota(jnp.int32, L)
    ibase = (lanes & 3) * 128 + (lanes >> 2)

    def chunk_body(k, carry):
        off = base + k * (K1_CHUNK * 512)
        pltpu.sync_copy(av_hbm.at[pl.ds(off, K1_CHUNK * 512)], in_v)

        def grp(g, c2):
            idx = ((g >> 5) * 512 + (g & 31) * 4) + ibase
            out_v[pl.ds(g * L, L)] = plsc.load_gather(in_v, [idx])
            return c2

        lax.fori_loop(0, K1_NGRP, grp, 0)
        pltpu.sync_copy(out_v, t_hbm.at[pl.ds(off, K1_CHUNK * 512)])
        return carry

    lax.fori_loop(0, K1_NCHUNK, chunk_body, 0)


CHUNK = 512
NCHUNK = PTS_PER_TILE // CHUNK
NGRP = CHUNK // L
GSLICE = 128
NGS = CHUNK // GSLICE
NTAB = 8
XY_OFF = (0, 128, 32768, 32896)


def _tile_body(pts_hbm, vg_hbm, out_hbm, pts_v, idx_v, fr_v, par_v, gat_v, out_v, sem):
    wid = lax.axis_index("s") * _NC + lax.axis_index("c")
    base = wid * PTS_PER_TILE
    lanes = lax.iota(jnp.int32, L)
    zero = jnp.zeros((L,), jnp.int32)

    def chunk_body(ci, carry):
        off = base + ci * CHUNK
        pltpu.sync_copy(pts_hbm.at[pl.ds(off * 3, CHUNK * 3)], pts_v)

        def idx_body(g, c2):
            s = g * L
            rows3 = (s + lanes) * 3
            px = plsc.load_gather(pts_v, [rows3])
            py = plsc.load_gather(pts_v, [rows3 + 1])
            pz = plsc.load_gather(pts_v, [rows3 + 2])
            gx = px * 127.5 + 127.5
            gy = py * 127.5 + 127.5
            gz = pz * 127.5 + 127.5
            x0 = gx.astype(jnp.int32)
            y0 = gy.astype(jnp.int32)
            z0 = gz.astype(jnp.int32)
            fx = gx - x0.astype(jnp.float32)
            fy = gy - y0.astype(jnp.float32)
            fz = gz - z0.astype(jnp.float32)
            x0 = jnp.minimum(jnp.maximum(x0, 0), GRID - 2)
            y0 = jnp.minimum(jnp.maximum(y0, 0), GRID - 2)
            z0 = jnp.minimum(jnp.maximum(z0, 0), GRID - 2)
            f = (x0 << 16) | (y0 << 8) | z0
            par = f & 1
            ra = f >> 1
            rb = ra + par
            sl = pl.ds(s, L)
            for t in range(4):
                idx_v[2 * t, sl] = ra + XY_OFF[t]
                idx_v[2 * t + 1, sl] = rb + XY_OFF[t]
            fr_v[0, sl] = fx
            fr_v[1, sl] = fy
            fr_v[2, sl] = fz
            par_v[sl] = par << 2
            return c2

        lax.fori_loop(0, NGRP, idx_body, 0)

        copies = []
        for t in range(NTAB):
            for j in range(NGS):
                copies.append(
                    pltpu.async_copy(
                        vg_hbm.at[idx_v.at[t, pl.ds(j * GSLICE, GSLICE)]],
                        gat_v.at[t, pl.ds(j * GSLICE, GSLICE)],
                        sem,
                    )
                )
        for cp in copies:
            cp.wait()

        def comb_body(g, c2):
            s = g * L
            rows = s + lanes
            sl = pl.ds(s, L)
            fx = fr_v[0, sl]
            fy = fr_v[1, sl]
            fz = fr_v[2, sl]
            par4 = par_v[sl]
            ex = 1.0 - fx
            ey = 1.0 - fy
            ez = 1.0 - fz
            wxy = (ex * ey, ex * fy, fx * ey, fx * fy)
            wa = tuple(w * ez for w in wxy)
            wb = tuple(w * fz for w in wxy)
            for ch in range(CH):
                sa = par4 + ch
                sb = (zero + (4 + ch)) - par4
                acc = wa[0] * plsc.load_gather(gat_v.at[0], [rows, sa])
                acc = acc + wb[0] * plsc.load_gather(gat_v.at[1], [rows, sb])
                for t in range(1, 4):
                    acc = acc + wa[t] * plsc.load_gather(gat_v.at[2 * t], [rows, sa])
                    acc = acc + wb[t] * plsc.load_gather(
                        gat_v.at[2 * t + 1], [rows, sb]
                    )
                o = ((g >> 3) * 4 + ch) * 128 + ((g & 7) * L)
                out_v[pl.ds(o, L)] = acc
            return c2

        lax.fori_loop(0, NGRP, comb_body, 0)
        pltpu.sync_copy(out_v, out_hbm.at[pl.ds(off * 4, CHUNK * 4)])
        return carry

    lax.fori_loop(0, NCHUNK, chunk_body, 0)


@jax.jit
def kernel(points, voxel_grid):
    av = voxel_grid.reshape(GRID, GRID, 2, 128, CH)
    av = av.transpose(0, 1, 2, 4, 3).reshape(NFLOAT)
    fmt = pl.kernel(
        _fmt_body,
        out_type=jax.ShapeDtypeStruct((NFLOAT,), jnp.float32),
        mesh=plsc.VectorSubcoreMesh(**_MESH),
        scratch_types=[
            pltpu.VMEM((K1_CHUNK * 512,), jnp.float32),
            pltpu.VMEM((K1_CHUNK * 512,), jnp.float32),
        ],
        compiler_params=_CP,
    )
    table = fmt(av).reshape(NFLOAT // 8, 8)

    pts = points.reshape(NPTS * 3)
    run = pl.kernel(
        _tile_body,
        out_type=jax.ShapeDtypeStruct((NPTS * 4,), jnp.float32),
        mesh=plsc.VectorSubcoreMesh(**_MESH),
        scratch_types=[
            pltpu.VMEM((CHUNK * 3,), jnp.float32),
            pltpu.VMEM((NTAB, CHUNK), jnp.int32),
            pltpu.VMEM((3, CHUNK), jnp.float32),
            pltpu.VMEM((CHUNK,), jnp.int32),
            pltpu.VMEM((NTAB, CHUNK, 2 * CH), jnp.float32),
            pltpu.VMEM((CHUNK * 4,), jnp.float32),
            pltpu.SemaphoreType.DMA,
        ],
        compiler_params=_CP,
    )
    out = run(pts, table)
    return out.reshape(NPTS // 128, CH, 128).transpose(0, 2, 1).reshape(NPTS, CH)


# speedup vs baseline: 13.0011x; 1.9684x over previous
import jax
import jax.numpy as jnp
from jax import lax
from jax.experimental import pallas as pl
from jax.experimental.pallas import tpu as pltpu
from jax.experimental.pallas import tpu_sc as plsc

GRID = 256
NPTS = 2097152
CH = 4
L = 16

_info = plsc.get_sparse_core_info()
_NC, _NS = _info.num_cores, _info.num_subcores
NW = _NC * _NS
PTS_PER_TILE = NPTS // NW

_CP = pltpu.CompilerParams(needs_layout_passes=False, use_tc_tiling_on_sc=False)
_MESH = dict(core_axis_name="c", subcore_axis_name="s")

NFLOAT = GRID * GRID * GRID * CH
NBLK = NFLOAT // 512
BLK_PER_TILE = NBLK // NW
K1_CHUNK = 64
K1_NCHUNK = BLK_PER_TILE // K1_CHUNK
K1_NGRP = K1_CHUNK * 32


def _fmt_body(av_hbm, t_hbm, in_v, out_v):
    wid = lax.axis_index("s") * _NC + lax.axis_index("c")
    base = wid * BLK_PER_TILE * 512
    lanes = lax.iota(jnp.int32, L)
    ibase = (lanes & 3) * 128 + (lanes >> 2)

    def chunk_body(k, carry):
        off = base + k * (K1_CHUNK * 512)
        pltpu.sync_copy(av_hbm.at[pl.ds(off, K1_CHUNK * 512)], in_v)

        def blk(b, c2):
            b512 = b * 512
            for i in range(32):
                idx = (b512 + i * 4) + ibase
                out_v[pl.ds(b512 + i * L, L)] = plsc.load_gather(in_v, [idx])
            return c2

        lax.fori_loop(0, K1_CHUNK, blk, 0)
        pltpu.sync_copy(out_v, t_hbm.at[pl.ds(off, K1_CHUNK * 512)])
        return carry

    lax.fori_loop(0, K1_NCHUNK, chunk_body, 0)


CHUNK = 512
NCHUNK = PTS_PER_TILE // CHUNK
NGRP = CHUNK // L
GSLICE = 128
NGS = CHUNK // GSLICE
NTAB = 8
XY_OFF = (0, 128, 32768, 32896)


def _tile_body(pts_hbm, vg_hbm, out_hbm, pts_v, idx_v, fr_v, par_v, gat_v, out_v, sem):
    wid = lax.axis_index("s") * _NC + lax.axis_index("c")
    base = wid * PTS_PER_TILE
    lanes = lax.iota(jnp.int32, L)
    zero = jnp.zeros((L,), jnp.int32)

    def chunk_body(ci, carry):
        off = base + ci * CHUNK
        pltpu.sync_copy(pts_hbm.at[pl.ds(off * 4, CHUNK * 4)], pts_v)

        def idx_body(g, c2):
            s = g * L
            pbase = (s >> 7) * 512 + (s & 127)
            px = pts_v[pl.ds(pbase, L)]
            py = pts_v[pl.ds(pbase + 128, L)]
            pz = pts_v[pl.ds(pbase + 256, L)]
            gx = px * 127.5 + 127.5
            gy = py * 127.5 + 127.5
            gz = pz * 127.5 + 127.5
            x0 = gx.astype(jnp.int32)
            y0 = gy.astype(jnp.int32)
            z0 = gz.astype(jnp.int32)
            fx = gx - x0.astype(jnp.float32)
            fy = gy - y0.astype(jnp.float32)
            fz = gz - z0.astype(jnp.float32)
            x0 = jnp.minimum(jnp.maximum(x0, 0), GRID - 2)
            y0 = jnp.minimum(jnp.maximum(y0, 0), GRID - 2)
            z0 = jnp.minimum(jnp.maximum(z0, 0), GRID - 2)
            f = (x0 << 16) | (y0 << 8) | z0
            par = f & 1
            ra = f >> 1
            rb = ra + par
            sl = pl.ds(s, L)
            for t in range(4):
                idx_v[2 * t, sl] = ra + XY_OFF[t]
                idx_v[2 * t + 1, sl] = rb + XY_OFF[t]
            fr_v[0, sl] = fx
            fr_v[1, sl] = fy
            fr_v[2, sl] = fz
            par_v[sl] = par << 2
            return c2

        lax.fori_loop(0, NGRP, idx_body, 0)

        copies = []
        for t in range(NTAB):
            for j in range(NGS):
                copies.append(
                    pltpu.async_copy(
                        vg_hbm.at[idx_v.at[t, pl.ds(j * GSLICE, GSLICE)]],
                        gat_v.at[t, pl.ds(j * GSLICE, GSLICE)],
                        sem,
                    )
                )
        for cp in copies:
            cp.wait()

        def comb_body(g, c2):
            s = g * L
            rows = s + lanes
            sl = pl.ds(s, L)
            fx = fr_v[0, sl]
            fy = fr_v[1, sl]
            fz = fr_v[2, sl]
            par4 = par_v[sl]
            ex = 1.0 - fx
            ey = 1.0 - fy
            ez = 1.0 - fz
            wxy = (ex * ey, ex * fy, fx * ey, fx * fy)
            wa = tuple(w * ez for w in wxy)
            wb = tuple(w * fz for w in wxy)
            for ch in range(CH):
                sa = par4 + ch
                sb = (zero + (4 + ch)) - par4
                acc = wa[0] * plsc.load_gather(gat_v.at[0], [rows, sa])
                acc = acc + wb[0] * plsc.load_gather(gat_v.at[1], [rows, sb])
                for t in range(1, 4):
                    acc = acc + wa[t] * plsc.load_gather(gat_v.at[2 * t], [rows, sa])
                    acc = acc + wb[t] * plsc.load_gather(
                        gat_v.at[2 * t + 1], [rows, sb]
                    )
                o = ((g >> 3) * 4 + ch) * 128 + ((g & 7) * L)
                out_v[pl.ds(o, L)] = acc
            return c2

        lax.fori_loop(0, NGRP, comb_body, 0)
        pltpu.sync_copy(out_v, out_hbm.at[pl.ds(off * 4, CHUNK * 4)])
        return carry

    lax.fori_loop(0, NCHUNK, chunk_body, 0)


@jax.jit
def kernel(points, voxel_grid):
    av = voxel_grid.reshape(GRID, GRID, 2, 128, CH)
    av = av.transpose(0, 1, 2, 4, 3).reshape(NFLOAT)
    fmt = pl.kernel(
        _fmt_body,
        out_type=jax.ShapeDtypeStruct((NFLOAT,), jnp.float32),
        mesh=plsc.VectorSubcoreMesh(**_MESH),
        scratch_types=[
            pltpu.VMEM((K1_CHUNK * 512,), jnp.float32),
            pltpu.VMEM((K1_CHUNK * 512,), jnp.float32),
        ],
        compiler_params=_CP,
    )
    table = fmt(av).reshape(NFLOAT // 8, 8)

    pp = jnp.pad(points, ((0, 0), (0, 1)))
    pts = pp.reshape(NPTS // 128, 128, 4).transpose(0, 2, 1).reshape(NPTS * 4)
    run = pl.kernel(
        _tile_body,
        out_type=jax.ShapeDtypeStruct((NPTS * 4,), jnp.float32),
        mesh=plsc.VectorSubcoreMesh(**_MESH),
        scratch_types=[
            pltpu.VMEM((CHUNK * 4,), jnp.float32),
            pltpu.VMEM((NTAB, CHUNK), jnp.int32),
            pltpu.VMEM((3, CHUNK), jnp.float32),
            pltpu.VMEM((CHUNK,), jnp.int32),
            pltpu.VMEM((NTAB, CHUNK, 2 * CH), jnp.float32),
            pltpu.VMEM((CHUNK * 4,), jnp.float32),
            pltpu.SemaphoreType.DMA,
        ],
        compiler_params=_CP,
    )
    out = run(pts, table)
    return out.reshape(NPTS // 128, CH, 128).transpose(0, 2, 1).reshape(NPTS, CH)
